# baseline (speedup 1.0000x reference)
import functools

import jax
import jax.numpy as jnp
from jax import lax
from jax.experimental import pallas as pl
from jax.experimental.pallas import tpu as pltpu
from jax.experimental.pallas import tpu_sc as plsc

BATCH = 16384
EMBED_DIM = 64
NUM_WORKERS = 32
B_PER_W = BATCH // NUM_WORKERS
HALF = B_PER_W // 2
GATHER_CHUNK = 128
GROUPS = HALF // 16

_mesh = plsc.VectorSubcoreMesh(core_axis_name="c", subcore_axis_name="s")


@functools.partial(
    pl.kernel,
    mesh=_mesh,
    compiler_params=pltpu.CompilerParams(needs_layout_passes=False),
    out_type=jax.ShapeDtypeStruct((BATCH,), jnp.float32),
    scratch_types=[
        pltpu.VMEM((B_PER_W,), jnp.int32),
        pltpu.VMEM((B_PER_W,), jnp.int32),
        pltpu.VMEM((HALF,), jnp.int32),
        pltpu.VMEM((HALF,), jnp.int32),
        pltpu.VMEM((HALF, 128), jnp.float32),
        pltpu.VMEM((HALF, 128), jnp.float32),
        pltpu.VMEM((B_PER_W,), jnp.float32),
        pltpu.SemaphoreType.DMA,
        pltpu.SemaphoreType.DMA,
    ],
)
def _dot_kernel(uid_hbm, iid_hbm, utab_hbm, itab_hbm, out_hbm,
                uidx_v, iidx_v, upair_v, ipair_v, urows_v, irows_v,
                out_v, sem_u, sem_i):
    wid = lax.axis_index("s") * 2 + lax.axis_index("c")
    base = wid * B_PER_W

    pltpu.sync_copy(uid_hbm.at[pl.ds(base, B_PER_W)], uidx_v)
    pltpu.sync_copy(iid_hbm.at[pl.ds(base, B_PER_W)], iidx_v)

    for h in range(2):
        def pair_body(k, carry):
            u = uidx_v[pl.ds(h * HALF + k * 16, 16)]
            i = iidx_v[pl.ds(h * HALF + k * 16, 16)]
            upair_v[pl.ds(k * 16, 16)] = lax.shift_right_logical(u, 1)
            ipair_v[pl.ds(k * 16, 16)] = lax.shift_right_logical(i, 1)
            return carry
        lax.fori_loop(0, GROUPS, pair_body, 0)

        copies = []
        for j in range(HALF // GATHER_CHUNK):
            sl = pl.ds(j * GATHER_CHUNK, GATHER_CHUNK)
            copies.append(pltpu.async_copy(
                utab_hbm.at[upair_v.at[sl]], urows_v.at[sl], sem_u))
            copies.append(pltpu.async_copy(
                itab_hbm.at[ipair_v.at[sl]], irows_v.at[sl], sem_i))
        for c in copies:
            c.wait()

        def group_body(g, carry):
            rid = g * 16 + lax.iota(jnp.int32, 16)
            u = uidx_v[pl.ds(h * HALF + g * 16, 16)]
            i = iidx_v[pl.ds(h * HALF + g * 16, 16)]
            ucol = lax.shift_left(jnp.bitwise_and(u, 1), 6)
            icol = lax.shift_left(jnp.bitwise_and(i, 1), 6)
            acc = jnp.zeros((16,), jnp.float32)
            for d in range(EMBED_DIM):
                uu = plsc.load_gather(urows_v, [rid, ucol + d])
                ii = plsc.load_gather(irows_v, [rid, icol + d])
                acc = acc + uu * ii
            out_v[pl.ds(h * HALF + g * 16, 16)] = acc
            return carry
        lax.fori_loop(0, GROUPS, group_body, 0)

    pltpu.sync_copy(out_v, out_hbm.at[pl.ds(base, B_PER_W)])


def kernel(inputs, user_table, item_table):
    user_ids = inputs[:, 0].astype(jnp.int32)
    item_ids = inputs[:, 1].astype(jnp.int32)
    return _dot_kernel(user_ids, item_ids,
                       user_table.reshape(500000, 128),
                       item_table.reshape(500000, 128))


# speedup vs baseline: 2.3905x; 2.3905x over previous
import functools

import jax
import jax.numpy as jnp
from jax import lax
from jax.experimental import pallas as pl
from jax.experimental.pallas import tpu as pltpu
from jax.experimental.pallas import tpu_sc as plsc

BATCH = 16384
EMBED_DIM = 64
NUM_WORKERS = 32
B_PER_W = BATCH // NUM_WORKERS
GROUP = 4
GROUPS = B_PER_W // GROUP
IDX_PAD = 16

_mesh = plsc.VectorSubcoreMesh(core_axis_name="c", subcore_axis_name="s")


@functools.partial(
    pl.kernel,
    mesh=_mesh,
    compiler_params=pltpu.CompilerParams(needs_layout_passes=False),
    out_type=jax.ShapeDtypeStruct((BATCH,), jnp.float32),
    scratch_types=[
        pltpu.VMEM((B_PER_W + IDX_PAD,), jnp.int32),
        pltpu.VMEM((B_PER_W + IDX_PAD,), jnp.int32),
        pltpu.VMEM((GROUP, EMBED_DIM, 128), jnp.float32),
        pltpu.VMEM((GROUP, EMBED_DIM, 128), jnp.float32),
        pltpu.VMEM((B_PER_W,), jnp.float32),
        pltpu.SemaphoreType.DMA,
        pltpu.SemaphoreType.DMA,
    ],
)
def _dot_kernel(uid_hbm, iid_hbm, utT_hbm, itT_hbm, out_hbm,
                uidx_v, iidx_v, uslab_v, islab_v, out_v, sem_u, sem_i):
    wid = lax.axis_index("s") * 2 + lax.axis_index("c")
    base = wid * B_PER_W

    pltpu.sync_copy(uid_hbm.at[pl.ds(base, B_PER_W)],
                    uidx_v.at[pl.ds(0, B_PER_W)])
    pltpu.sync_copy(iid_hbm.at[pl.ds(base, B_PER_W)],
                    iidx_v.at[pl.ds(0, B_PER_W)])

    zeros16 = jnp.zeros((16,), jnp.float32)

    def zero_body(i, carry):
        out_v[pl.ds(i * 16, 16)] = zeros16
        return carry
    lax.fori_loop(0, B_PER_W // 16, zero_body, 0)

    lane = lax.iota(jnp.int32, 16)
    row_of_lane = jnp.bitwise_and(lane, GROUP - 1)
    qblk = lax.shift_right_logical(lane, 2)

    def group_body(g, carry):
        uvec = uidx_v[pl.ds(g * GROUP, 16)]
        ivec = iidx_v[pl.ds(g * GROUP, 16)]
        copies = []
        for l in range(GROUP):
            ua = pl.multiple_of(
                lax.shift_left(lax.shift_right_logical(uvec[l], 7), 7), 128)
            ia = pl.multiple_of(
                lax.shift_left(lax.shift_right_logical(ivec[l], 7), 7), 128)
            copies.append(pltpu.async_copy(
                utT_hbm.at[:, pl.ds(ua, 128)], uslab_v.at[l], sem_u))
            copies.append(pltpu.async_copy(
                itT_hbm.at[:, pl.ds(ia, 128)], islab_v.at[l], sem_i))
        for c in copies:
            c.wait()

        um16 = jnp.bitwise_and(
            plsc.load_gather(uidx_v, [g * GROUP + row_of_lane]), 127)
        im16 = jnp.bitwise_and(
            plsc.load_gather(iidx_v, [g * GROUP + row_of_lane]), 127)
        acc = jnp.zeros((16,), jnp.float32)
        for j in range(16):
            dv = lax.shift_left(qblk, 4) + j
            uu = plsc.load_gather(uslab_v, [row_of_lane, dv, um16])
            ii = plsc.load_gather(islab_v, [row_of_lane, dv, im16])
            acc = acc + uu * ii
        plsc.addupdate_scatter(out_v, [g * GROUP + row_of_lane], acc)
        return carry

    lax.fori_loop(0, GROUPS, group_body, 0)

    pltpu.sync_copy(out_v, out_hbm.at[pl.ds(base, B_PER_W)])


def kernel(inputs, user_table, item_table):
    user_ids = inputs[:, 0].astype(jnp.int32)
    item_ids = inputs[:, 1].astype(jnp.int32)
    return _dot_kernel(user_ids, item_ids, user_table.T, item_table.T)


# speedup vs baseline: 2.6028x; 1.0888x over previous
import functools

import jax
import jax.numpy as jnp
from jax import lax
from jax.experimental import pallas as pl
from jax.experimental.pallas import tpu as pltpu
from jax.experimental.pallas import tpu_sc as plsc

BATCH = 16384
EMBED_DIM = 64
NUM_WORKERS = 32
B_PER_W = BATCH // NUM_WORKERS
GROUP = 2
GROUPS = B_PER_W // GROUP
IDX_PAD = 16

_mesh = plsc.VectorSubcoreMesh(core_axis_name="c", subcore_axis_name="s")


@functools.partial(
    pl.kernel,
    mesh=_mesh,
    compiler_params=pltpu.CompilerParams(needs_layout_passes=False),
    out_type=jax.ShapeDtypeStruct((BATCH,), jnp.float32),
    scratch_types=[
        pltpu.VMEM((B_PER_W + IDX_PAD,), jnp.int32),
        pltpu.VMEM((B_PER_W + IDX_PAD,), jnp.int32),
        pltpu.VMEM((2, GROUP, EMBED_DIM, 128), jnp.float32),
        pltpu.VMEM((2, GROUP, EMBED_DIM, 128), jnp.float32),
        pltpu.VMEM((B_PER_W,), jnp.float32),
        pltpu.SemaphoreType.DMA,
        pltpu.SemaphoreType.DMA,
        pltpu.SemaphoreType.DMA,
        pltpu.SemaphoreType.DMA,
    ],
)
def _dot_kernel(uid_hbm, iid_hbm, utT_hbm, itT_hbm, out_hbm,
                uidx_v, iidx_v, uslab_v, islab_v, out_v,
                sem_u0, sem_i0, sem_u1, sem_i1):
    wid = lax.axis_index("s") * 2 + lax.axis_index("c")
    base = wid * B_PER_W
    sems = ((sem_u0, sem_i0), (sem_u1, sem_i1))

    pltpu.sync_copy(uid_hbm.at[pl.ds(base, B_PER_W)],
                    uidx_v.at[pl.ds(0, B_PER_W)])
    pltpu.sync_copy(iid_hbm.at[pl.ds(base, B_PER_W)],
                    iidx_v.at[pl.ds(0, B_PER_W)])

    zeros16 = jnp.zeros((16,), jnp.float32)

    def zero_body(i, carry):
        out_v[pl.ds(i * 16, 16)] = zeros16
        return carry
    lax.fori_loop(0, B_PER_W // 16, zero_body, 0)

    lane = lax.iota(jnp.int32, 16)
    row_of_lane = jnp.bitwise_and(lane, GROUP - 1)
    qblk = lax.shift_right_logical(lane, 1)

    def issue(phase, g):
        uvec = uidx_v[pl.ds(g * GROUP, 16)]
        ivec = iidx_v[pl.ds(g * GROUP, 16)]
        for l in range(GROUP):
            ua = pl.multiple_of(
                lax.shift_left(lax.shift_right_logical(uvec[l], 7), 7), 128)
            ia = pl.multiple_of(
                lax.shift_left(lax.shift_right_logical(ivec[l], 7), 7), 128)
            pltpu.async_copy(utT_hbm.at[:, pl.ds(ua, 128)],
                             uslab_v.at[phase, l], sems[phase][0])
            pltpu.async_copy(itT_hbm.at[:, pl.ds(ia, 128)],
                             islab_v.at[phase, l], sems[phase][1])

    def wait_phase(phase):
        for l in range(GROUP):
            pltpu.make_async_copy(utT_hbm.at[:, pl.ds(0, 128)],
                                  uslab_v.at[phase, l],
                                  sems[phase][0]).wait()
            pltpu.make_async_copy(itT_hbm.at[:, pl.ds(0, 128)],
                                  islab_v.at[phase, l],
                                  sems[phase][1]).wait()

    def compute(phase, g):
        wait_phase(phase)
        um16 = jnp.bitwise_and(
            plsc.load_gather(uidx_v, [g * GROUP + row_of_lane]), 127)
        im16 = jnp.bitwise_and(
            plsc.load_gather(iidx_v, [g * GROUP + row_of_lane]), 127)
        acc = jnp.zeros((16,), jnp.float32)
        for j in range(EMBED_DIM // 8):
            dv = lax.shift_left(qblk, 3) + j
            uu = plsc.load_gather(uslab_v.at[phase], [row_of_lane, dv, um16])
            ii = plsc.load_gather(islab_v.at[phase], [row_of_lane, dv, im16])
            acc = acc + uu * ii
        plsc.addupdate_scatter(out_v, [g * GROUP + row_of_lane], acc)

    def body(p, carry):
        g0 = p * 2
        issue(1, g0 + 1)
        compute(0, g0)
        issue(0, jnp.minimum(g0 + 2, GROUPS - 2))
        compute(1, g0 + 1)
        return carry

    issue(0, 0)
    lax.fori_loop(0, GROUPS // 2, body, 0)
    wait_phase(0)

    pltpu.sync_copy(out_v, out_hbm.at[pl.ds(base, B_PER_W)])


def kernel(inputs, user_table, item_table):
    user_ids = inputs[:, 0].astype(jnp.int32)
    item_ids = inputs[:, 1].astype(jnp.int32)
    return _dot_kernel(user_ids, item_ids, user_table.T, item_table.T)


# speedup vs baseline: 2.6052x; 1.0009x over previous
import functools

import jax
import jax.numpy as jnp
from jax import lax
from jax.experimental import pallas as pl
from jax.experimental.pallas import tpu as pltpu
from jax.experimental.pallas import tpu_sc as plsc

BATCH = 16384
EMBED_DIM = 64
NUM_WORKERS = 32
B_PER_W = BATCH // NUM_WORKERS
GROUP = 2
GROUPS = B_PER_W // GROUP
IDX_PAD = 16

_mesh = plsc.VectorSubcoreMesh(core_axis_name="c", subcore_axis_name="s")


@functools.partial(
    pl.kernel,
    mesh=_mesh,
    compiler_params=pltpu.CompilerParams(needs_layout_passes=False),
    out_type=jax.ShapeDtypeStruct((BATCH,), jnp.float32),
    scratch_types=[
        pltpu.VMEM((B_PER_W + IDX_PAD,), jnp.int32),
        pltpu.VMEM((B_PER_W + IDX_PAD,), jnp.int32),
        pltpu.VMEM((2, GROUP, EMBED_DIM, 128), jnp.float32),
        pltpu.VMEM((2, GROUP, EMBED_DIM, 128), jnp.float32),
        pltpu.VMEM((B_PER_W,), jnp.float32),
        pltpu.SemaphoreType.DMA,
        pltpu.SemaphoreType.DMA,
        pltpu.SemaphoreType.DMA,
        pltpu.SemaphoreType.DMA,
    ],
)
def _dot_kernel(uid_hbm, iid_hbm, utT_hbm, itT_hbm, out_hbm,
                uidx_v, iidx_v, uslab_v, islab_v, out_v,
                sem_u0, sem_i0, sem_u1, sem_i1):
    wid = lax.axis_index("s") * 2 + lax.axis_index("c")
    base = wid * B_PER_W
    sems = ((sem_u0, sem_i0), (sem_u1, sem_i1))

    pltpu.sync_copy(uid_hbm.at[pl.ds(base, B_PER_W)],
                    uidx_v.at[pl.ds(0, B_PER_W)])
    pltpu.sync_copy(iid_hbm.at[pl.ds(base, B_PER_W)],
                    iidx_v.at[pl.ds(0, B_PER_W)])

    zeros16 = jnp.zeros((16,), jnp.float32)

    def zero_body(i, carry):
        out_v[pl.ds(i * 16, 16)] = zeros16
        return carry
    lax.fori_loop(0, B_PER_W // 16, zero_body, 0)

    lane = lax.iota(jnp.int32, 16)
    row_of_lane = jnp.bitwise_and(lane, GROUP - 1)
    qblk = lax.shift_right_logical(lane, 1)

    def issue(phase, g):
        uvec = uidx_v[pl.ds(g * GROUP, 16)]
        ivec = iidx_v[pl.ds(g * GROUP, 16)]
        for l in range(GROUP):
            ua = pl.multiple_of(
                lax.shift_left(lax.shift_right_logical(uvec[l], 7), 7), 128)
            ia = pl.multiple_of(
                lax.shift_left(lax.shift_right_logical(ivec[l], 7), 7), 128)
            for h in range(2):
                cs = pl.ds(h * 32, 32)
                pltpu.async_copy(utT_hbm.at[cs, pl.ds(ua, 128)],
                                 uslab_v.at[phase, l, cs], sems[phase][0])
                pltpu.async_copy(itT_hbm.at[cs, pl.ds(ia, 128)],
                                 islab_v.at[phase, l, cs], sems[phase][1])

    def wait_phase(phase):
        for l in range(GROUP):
            pltpu.make_async_copy(utT_hbm.at[:, pl.ds(0, 128)],
                                  uslab_v.at[phase, l],
                                  sems[phase][0]).wait()
            pltpu.make_async_copy(itT_hbm.at[:, pl.ds(0, 128)],
                                  islab_v.at[phase, l],
                                  sems[phase][1]).wait()

    def compute(phase, g):
        wait_phase(phase)
        um16 = jnp.bitwise_and(
            plsc.load_gather(uidx_v, [g * GROUP + row_of_lane]), 127)
        im16 = jnp.bitwise_and(
            plsc.load_gather(iidx_v, [g * GROUP + row_of_lane]), 127)
        acc = jnp.zeros((16,), jnp.float32)
        for j in range(EMBED_DIM // 8):
            dv = lax.shift_left(qblk, 3) + j
            uu = plsc.load_gather(uslab_v.at[phase], [row_of_lane, dv, um16])
            ii = plsc.load_gather(islab_v.at[phase], [row_of_lane, dv, im16])
            acc = acc + uu * ii
        plsc.addupdate_scatter(out_v, [g * GROUP + row_of_lane], acc)

    def body(p, carry):
        g0 = p * 2
        issue(1, g0 + 1)
        compute(0, g0)
        issue(0, jnp.minimum(g0 + 2, GROUPS - 2))
        compute(1, g0 + 1)
        return carry

    issue(0, 0)
    lax.fori_loop(0, GROUPS // 2, body, 0)
    wait_phase(0)

    pltpu.sync_copy(out_v, out_hbm.at[pl.ds(base, B_PER_W)])


def kernel(inputs, user_table, item_table):
    user_ids = inputs[:, 0].astype(jnp.int32)
    item_ids = inputs[:, 1].astype(jnp.int32)
    return _dot_kernel(user_ids, item_ids, user_table.T, item_table.T)
